# baseline (speedup 1.0000x reference)
import jax
import jax.numpy as jnp
from jax import lax
from jax.experimental import pallas as pl
from jax.experimental.pallas import tpu as pltpu

_N_CLUSTERS = 50
_CLUSTER = 2000
_TOT = 3200
_T = 128
_D = 8
_K = 64
_CPB = 5
_NBLK = _N_CLUSTERS // _CPB
_KB = _CPB * _K


def _step(x_ref, t_ref, sgf_ref, dgf_ref, slf_ref, dl_ref, y_ref, olog):
    i = pl.program_id(0)

    @pl.when(i == 0)
    def _():
        olog[...] = jnp.zeros_like(olog)

    tm = t_ref[...]
    sgf = sgf_ref[...]

    for j in range(_CPB):
        xb = x_ref[0, pl.ds(j * _CLUSTER, _CLUSTER), :]

        m = (dgf_ref[0, pl.ds(j * _K, _K), :] == sgf).astype(jnp.bfloat16)
        inc = jnp.dot(m, olog[...], preferred_element_type=jnp.float32)
        convinc = jnp.dot(inc.astype(jnp.bfloat16), tm,
                          preferred_element_type=jnp.float32)

        yb = jnp.dot(xb.astype(jnp.bfloat16), tm,
                     preferred_element_type=jnp.float32)
        dl_c = dl_ref[0, :, pl.ds(j * _K, _K)]
        ohd = (lax.broadcasted_iota(jnp.int32, (_CLUSTER, _K), 0)
               == dl_c).astype(jnp.bfloat16)
        y = yb + jnp.dot(ohd, convinc.astype(jnp.bfloat16),
                         preferred_element_type=jnp.float32)
        y_ref[0, pl.ds(j * _CLUSTER, _CLUSTER), :] = y

        ohs = (lax.broadcasted_iota(jnp.int32, (_K, _CLUSTER), 1)
               == slf_ref[0, pl.ds(j * _K, _K), :]).astype(jnp.bfloat16)
        out = jnp.dot(ohs, y, preferred_element_type=jnp.float32)
        olog[pl.ds(i * _KB + j * _K, _K), :] = out.astype(jnp.bfloat16)


def _toeplitz(fir):
    idx = jnp.arange(_T)
    diff = idx[None, :] - idx[:, None]
    mask = (diff >= 0) & (diff < _D)
    return jnp.where(mask, fir[jnp.clip(diff, 0, _D - 1)], 0.0)


def kernel(x, kernel, dst_local, dst_gidx, src_local, src_gidx):
    fir = kernel
    tmat = _toeplitz(fir).astype(jnp.bfloat16)

    sgf = src_gidx.astype(jnp.int32).reshape(1, _TOT)
    dgf = dst_gidx.astype(jnp.int32).reshape(_NBLK, _KB, 1)
    slf = src_local.astype(jnp.int32).reshape(_NBLK, _KB, 1)
    dl3 = dst_local.astype(jnp.int32).reshape(_NBLK, 1, _KB)

    y = pl.pallas_call(
        _step,
        grid=(_NBLK,),
        in_specs=[
            pl.BlockSpec((1, _CPB * _CLUSTER, _T), lambda i: (0, i, 0)),
            pl.BlockSpec((_T, _T), lambda i: (0, 0)),
            pl.BlockSpec((1, _TOT), lambda i: (0, 0)),
            pl.BlockSpec((1, _KB, 1), lambda i: (i, 0, 0)),
            pl.BlockSpec((1, _KB, 1), lambda i: (i, 0, 0)),
            pl.BlockSpec((1, 1, _KB), lambda i: (i, 0, 0)),
        ],
        out_specs=pl.BlockSpec((1, _CPB * _CLUSTER, _T), lambda i: (0, i, 0)),
        out_shape=jax.ShapeDtypeStruct(x.shape, jnp.float32),
        scratch_shapes=[pltpu.VMEM((_TOT, _T), jnp.bfloat16)],
        compiler_params=pltpu.CompilerParams(
            dimension_semantics=("arbitrary",),
        ),
    )(x, tmat, sgf, dgf, slf, dl3)
    return y


# speedup vs baseline: 1.3926x; 1.3926x over previous
import jax
import jax.numpy as jnp
from jax import lax
from jax.experimental import pallas as pl
from jax.experimental.pallas import tpu as pltpu

_N_CLUSTERS = 50
_CLUSTER = 2000
_TOT = 3200
_T = 128
_D = 8
_K = 64
_CPB = 2
_NBLK = _N_CLUSTERS // _CPB
_KB = _CPB * _K


def _step(x_ref, t_ref, sgf_ref, dl_ref, dg_ref, sl_ref, y_ref, olog):
    i = pl.program_id(0)

    @pl.when(i == 0)
    def _():
        olog[...] = jnp.zeros_like(olog)

    tm = t_ref[...]
    sgf = sgf_ref[0, :]

    for j in range(_CPB):
        xb = x_ref[0, pl.ds(j * _CLUSTER, _CLUSTER), :]
        dl = dl_ref[0, j, :]
        dg = dg_ref[0, j, :]
        sl = sl_ref[0, j, :]

        m = (dg[:, None] == sgf[None, :]).astype(jnp.bfloat16)
        inc = jnp.dot(m, olog[...], preferred_element_type=jnp.float32)

        ohd = (lax.broadcasted_iota(jnp.int32, (_CLUSTER, _K), 0)
               == dl[None, :]).astype(jnp.bfloat16)
        z = xb + jnp.dot(ohd, inc, preferred_element_type=jnp.float32)
        y = jnp.dot(z, tm, preferred_element_type=jnp.float32)
        y_ref[0, pl.ds(j * _CLUSTER, _CLUSTER), :] = y

        ohs = (lax.broadcasted_iota(jnp.int32, (_K, _CLUSTER), 1)
               == sl[:, None]).astype(jnp.bfloat16)
        out = jnp.dot(ohs, y, preferred_element_type=jnp.float32)
        olog[pl.ds(i * _KB + j * _K, _K), :] = out.astype(jnp.bfloat16)


def _toeplitz(fir):
    idx = jnp.arange(_T)
    diff = idx[None, :] - idx[:, None]
    mask = (diff >= 0) & (diff < _D)
    return jnp.where(mask, fir[jnp.clip(diff, 0, _D - 1)], 0.0).astype(jnp.float32)


def kernel(x, kernel, dst_local, dst_gidx, src_local, src_gidx):
    fir = kernel
    tmat = _toeplitz(fir)

    sgf = src_gidx.astype(jnp.int32).reshape(1, _TOT)
    dl3 = dst_local.astype(jnp.int32).reshape(_NBLK, _CPB, _K)
    dg3 = dst_gidx.astype(jnp.int32).reshape(_NBLK, _CPB, _K)
    sl3 = src_local.astype(jnp.int32).reshape(_NBLK, _CPB, _K)

    y = pl.pallas_call(
        _step,
        grid=(_NBLK,),
        in_specs=[
            pl.BlockSpec((1, _CPB * _CLUSTER, _T), lambda i: (0, i, 0)),
            pl.BlockSpec((_T, _T), lambda i: (0, 0)),
            pl.BlockSpec((1, _TOT), lambda i: (0, 0)),
            pl.BlockSpec((1, _CPB, _K), lambda i: (i, 0, 0)),
            pl.BlockSpec((1, _CPB, _K), lambda i: (i, 0, 0)),
            pl.BlockSpec((1, _CPB, _K), lambda i: (i, 0, 0)),
        ],
        out_specs=pl.BlockSpec((1, _CPB * _CLUSTER, _T), lambda i: (0, i, 0)),
        out_shape=jax.ShapeDtypeStruct(x.shape, jnp.float32),
        scratch_shapes=[pltpu.VMEM((_TOT, _T), jnp.bfloat16)],
        compiler_params=pltpu.CompilerParams(
            dimension_semantics=("arbitrary",),
        ),
    )(x, tmat, sgf, dl3, dg3, sl3)
    return y
